# baseline (speedup 1.0000x reference)
import dataclasses
import functools

import jax
import jax.numpy as jnp
from jax import lax
from jax.experimental import pallas as pl
from jax.experimental.pallas import tpu as pltpu
from jax.experimental.pallas import tpu_sc as plsc

N = 10000
F = 128
OUT_F = 64
E = 320000
SLOPE = (1.0 / 8.0 + 1.0 / 3.0) / 2.0

NC, NS = 2, 16
NW = NC * NS
CHUNK = 80
NCH = 128
NU = NCH // 4
E_PAD = NW * NCH * CHUNK
TOT_CH = E_PAD // CHUNK
ZB = 128
Z_ROWS = 10240
RPT = Z_ROWS // NS

_MESH = dict(core_axis_name="c", subcore_axis_name="s")


NBLK = Z_ROWS // ZB


def _sc_deg(idx_hbm):
    cp = pltpu.CompilerParams()
    if "needs_layout_passes" in pltpu.CompilerParams.__dataclass_fields__:
        cp = dataclasses.replace(cp, needs_layout_passes=False)

    @functools.partial(
        pl.kernel,
        out_type=jax.ShapeDtypeStruct((NC, Z_ROWS), jnp.float32),
        mesh=plsc.VectorSubcoreMesh(**_MESH),
        compiler_params=cp,
        scratch_types=[
            pltpu.VMEM((NCH, 2, CHUNK), jnp.int32),
            pltpu.VMEM((Z_ROWS,), jnp.float32),
            pltpu.VMEM((NS, ZB), jnp.float32),
            pltpu.VMEM_SHARED((NS, Z_ROWS), jnp.float32),
        ],
    )
    def deg_kernel(idx_ref, out_ref, idxv, hist, sumb, parts):
        c = lax.axis_index("c")
        s = lax.axis_index("s")
        wid = s * NC + c

        @pl.loop(0, Z_ROWS, step=16)
        def _(i):
            hist[pl.ds(i, 16)] = jnp.zeros((16,), jnp.float32)

        pltpu.sync_copy(idx_ref.at[pl.ds(wid * NCH, NCH)], idxv)
        ones = jnp.ones((16,), jnp.float32)

        @pl.loop(0, NCH)
        def _(j):
            for l in range(CHUNK // 16):
                idx = idxv[j, 1, pl.ds(l * 16, 16)]
                plsc.addupdate_scatter(hist, [idx], ones)

        pltpu.sync_copy(hist, parts.at[s])
        plsc.subcore_barrier()

        for k in range(NBLK // NS):
            blk = s + NS * k
            col0 = blk * ZB
            pltpu.sync_copy(parts.at[:, pl.ds(col0, ZB)], sumb)

            @pl.loop(0, ZB, step=16)
            def _(i):
                v = sumb[0, pl.ds(i, 16)]
                for t in range(1, NS):
                    v = v + sumb[t, pl.ds(i, 16)]
                hist[pl.ds(i, 16)] = v

            pltpu.sync_copy(hist.at[pl.ds(0, ZB)],
                            out_ref.at[c, pl.ds(col0, ZB)])

    return deg_kernel(idx_hbm)


def _sc_scatter(y, idx_hbm, zeros128):
    @functools.partial(
        pl.kernel,
        out_type=jax.ShapeDtypeStruct((NC, Z_ROWS, F), jnp.float32),
        mesh=plsc.VectorSubcoreMesh(**_MESH),
        scratch_types=[
            pltpu.VMEM((4, 2, CHUNK), jnp.int32),
            pltpu.VMEM((4, CHUNK, F), jnp.float32),
            pltpu.VMEM_SHARED((Z_ROWS, F), jnp.float32),
            [pltpu.SemaphoreType.DMA] * 4,
            [pltpu.SemaphoreType.DMA] * 4,
        ],
    )
    def scat_kernel(y_ref, idx_ref, zeros_ref, out_ref, idxg, rows, acc,
                    semg, sems):
        c = lax.axis_index("c")
        s = lax.axis_index("s")
        wid = s * NC + c
        row0 = s * RPT
        for k in range(RPT // ZB):
            pltpu.sync_copy(zeros_ref, acc.at[pl.ds(row0 + k * ZB, ZB)])
        plsc.subcore_barrier()

        grp0 = wid * NCH

        @pl.loop(0, NU)
        def _(g):
            pltpu.sync_copy(idx_ref.at[pl.ds(grp0 + g * 4, 4)], idxg)
            gat = [pltpu.async_copy(y_ref.at[idxg.at[i, 0]],
                                    rows.at[i], semg[i])
                   for i in range(4)]
            sca = []
            for i in range(4):
                gat[i].wait()
                sca.append(pltpu.async_copy(rows.at[i], acc.at[idxg.at[i, 1]],
                                            sems[i], add=True))
            for d in sca:
                d.wait()

        plsc.subcore_barrier()
        pltpu.sync_copy(acc.at[pl.ds(row0, RPT)], out_ref.at[c, pl.ds(row0, RPT)])

    return scat_kernel(y, idx_hbm, zeros128)


def _dinv(dpc):
    return lax.rsqrt(dpc[0] + dpc[1] + 1.0)


def _dot(a, b):
    return jnp.dot(a, b, preferred_element_type=jnp.float32,
                   precision=lax.Precision.HIGHEST)


def _tc_xw(x, W):
    def body(x_ref, w_ref, o_ref):
        o_ref[...] = _dot(x_ref[...], w_ref[...])

    return pl.pallas_call(
        body, out_shape=jax.ShapeDtypeStruct((Z_ROWS, F), jnp.float32),
    )(x, W)


def _tc_scale(xw, dp):
    def body(xw_ref, dp_ref, o_ref):
        o_ref[...] = _dinv(dp_ref[...]) * xw_ref[...]

    return pl.pallas_call(
        body, out_shape=jax.ShapeDtypeStruct((Z_ROWS, F), jnp.float32),
    )(xw, dp)


def _tc_mid(zp, y, b, W, dp):
    def body(zp_ref, y_ref, b_ref, w_ref, dp_ref, o_ref):
        dinv = _dinv(dp_ref[...])
        h = dinv * (zp_ref[0] + zp_ref[1] + y_ref[...]) + b_ref[...]
        h = jnp.where(h >= 0, h, h * SLOPE)
        o_ref[...] = dinv * _dot(h, w_ref[...])

    return pl.pallas_call(
        body, out_shape=jax.ShapeDtypeStruct((Z_ROWS, F), jnp.float32),
    )(zp, y, b, W, dp)


def _tc_final(zp, y, b, lw, lb, dp):
    def body(zp_ref, y_ref, b_ref, lw_ref, lb_ref, dp_ref, o_ref):
        dinv = _dinv(dp_ref[...])
        h = dinv * (zp_ref[0] + zp_ref[1] + y_ref[...]) + b_ref[...]
        h = jnp.where(h >= 0, h, h * SLOPE)
        o_ref[...] = jax.nn.sigmoid(_dot(h, lw_ref[...]) + lb_ref[...])

    return pl.pallas_call(
        body, out_shape=jax.ShapeDtypeStruct((Z_ROWS, OUT_F), jnp.float32),
    )(zp, y, b, lw, lb, dp)


def kernel(x, edge_index, W1, b1, W2, b2, lin_W, lin_b):
    src = edge_index[0].astype(jnp.int32)
    dst = edge_index[1].astype(jnp.int32)
    pad = N + jnp.arange(E_PAD - E, dtype=jnp.int32) % (Z_ROWS - N)
    src_p = jnp.concatenate([src, pad]).reshape(TOT_CH, CHUNK)
    dst_p = jnp.concatenate([dst, pad]).reshape(TOT_CH, CHUNK)
    idx_hbm = jnp.stack([src_p, dst_p], axis=1)

    x_p = jnp.pad(x, ((0, Z_ROWS - N), (0, 0)))
    zeros128 = jnp.zeros((ZB, F), jnp.float32)
    b1r = b1.reshape(1, F)
    b2r = b2.reshape(1, F)
    lbr = lin_b.reshape(1, OUT_F)

    dpc = _sc_deg(idx_hbm).reshape(NC, Z_ROWS, 1)
    xw1 = _tc_xw(x_p, W1)
    y1 = _tc_scale(xw1, dpc)
    zp1 = _sc_scatter(y1, idx_hbm, zeros128)
    y2 = _tc_mid(zp1, y1, b1r, W2, dpc)
    zp2 = _sc_scatter(y2, idx_hbm, zeros128)
    out = _tc_final(zp2, y2, b2r, lin_W, lbr, dpc)
    return out[:N]


# speedup vs baseline: 1.0184x; 1.0184x over previous
import dataclasses
import functools

import jax
import jax.numpy as jnp
from jax import lax
from jax.experimental import pallas as pl
from jax.experimental.pallas import tpu as pltpu
from jax.experimental.pallas import tpu_sc as plsc

N = 10000
F = 128
OUT_F = 64
E = 320000
SLOPE = (1.0 / 8.0 + 1.0 / 3.0) / 2.0

NC, NS = 2, 16
NW = NC * NS
CHUNK = 112
G = 3
NCH = 90
NU = NCH // G
E_PAD = NW * NCH * CHUNK
TOT_CH = E_PAD // CHUNK
ZB = 128
Z_ROWS = 10240
RPT = Z_ROWS // NS

_MESH = dict(core_axis_name="c", subcore_axis_name="s")


NBLK = Z_ROWS // ZB


def _sc_deg(idx_hbm):
    cp = pltpu.CompilerParams()
    if "needs_layout_passes" in pltpu.CompilerParams.__dataclass_fields__:
        cp = dataclasses.replace(cp, needs_layout_passes=False)

    @functools.partial(
        pl.kernel,
        out_type=jax.ShapeDtypeStruct((NC, Z_ROWS), jnp.float32),
        mesh=plsc.VectorSubcoreMesh(**_MESH),
        compiler_params=cp,
        scratch_types=[
            pltpu.VMEM((NCH, 2, CHUNK), jnp.int32),
            pltpu.VMEM((Z_ROWS,), jnp.float32),
            pltpu.VMEM((NS, ZB), jnp.float32),
            pltpu.VMEM_SHARED((NS, Z_ROWS), jnp.float32),
        ],
    )
    def deg_kernel(idx_ref, out_ref, idxv, hist, sumb, parts):
        c = lax.axis_index("c")
        s = lax.axis_index("s")
        wid = s * NC + c

        @pl.loop(0, Z_ROWS, step=16)
        def _(i):
            hist[pl.ds(i, 16)] = jnp.zeros((16,), jnp.float32)

        pltpu.sync_copy(idx_ref.at[pl.ds(wid * NCH, NCH)], idxv)
        ones = jnp.ones((16,), jnp.float32)

        @pl.loop(0, NCH)
        def _(j):
            for l in range(CHUNK // 16):
                idx = idxv[j, 1, pl.ds(l * 16, 16)]
                plsc.addupdate_scatter(hist, [idx], ones)

        pltpu.sync_copy(hist, parts.at[s])
        plsc.subcore_barrier()

        for k in range(NBLK // NS):
            blk = s + NS * k
            col0 = blk * ZB
            pltpu.sync_copy(parts.at[:, pl.ds(col0, ZB)], sumb)

            @pl.loop(0, ZB, step=16)
            def _(i):
                v = sumb[0, pl.ds(i, 16)]
                for t in range(1, NS):
                    v = v + sumb[t, pl.ds(i, 16)]
                hist[pl.ds(i, 16)] = v

            pltpu.sync_copy(hist.at[pl.ds(0, ZB)],
                            out_ref.at[c, pl.ds(col0, ZB)])

    return deg_kernel(idx_hbm)


def _sc_scatter(y, idx_hbm, zeros128):
    @functools.partial(
        pl.kernel,
        out_type=jax.ShapeDtypeStruct((NC, Z_ROWS, F), jnp.float32),
        mesh=plsc.VectorSubcoreMesh(**_MESH),
        scratch_types=[
            pltpu.VMEM((G, 2, CHUNK), jnp.int32),
            pltpu.VMEM((G, CHUNK, F), jnp.float32),
            pltpu.VMEM_SHARED((Z_ROWS, F), jnp.float32),
            [pltpu.SemaphoreType.DMA] * G,
            [pltpu.SemaphoreType.DMA] * G,
        ],
    )
    def scat_kernel(y_ref, idx_ref, zeros_ref, out_ref, idxg, rows, acc,
                    semg, sems):
        c = lax.axis_index("c")
        s = lax.axis_index("s")
        wid = s * NC + c
        row0 = s * RPT
        for k in range(RPT // ZB):
            pltpu.sync_copy(zeros_ref, acc.at[pl.ds(row0 + k * ZB, ZB)])
        plsc.subcore_barrier()

        grp0 = wid * NCH

        @pl.loop(0, NU)
        def _(g):
            pltpu.sync_copy(idx_ref.at[pl.ds(grp0 + g * G, G)], idxg)
            gat = [pltpu.async_copy(y_ref.at[idxg.at[i, 0]],
                                    rows.at[i], semg[i])
                   for i in range(G)]
            sca = []
            for i in range(G):
                gat[i].wait()
                sca.append(pltpu.async_copy(rows.at[i], acc.at[idxg.at[i, 1]],
                                            sems[i], add=True))
            for d in sca:
                d.wait()

        plsc.subcore_barrier()
        pltpu.sync_copy(acc.at[pl.ds(row0, RPT)], out_ref.at[c, pl.ds(row0, RPT)])

    return scat_kernel(y, idx_hbm, zeros128)


def _dinv(dpc):
    return lax.rsqrt(dpc[0] + dpc[1] + 1.0)


def _dot(a, b):
    return jnp.dot(a, b, preferred_element_type=jnp.float32,
                   precision=lax.Precision.HIGHEST)


def _tc_xw(x, W):
    def body(x_ref, w_ref, o_ref):
        o_ref[...] = _dot(x_ref[...], w_ref[...])

    return pl.pallas_call(
        body, out_shape=jax.ShapeDtypeStruct((Z_ROWS, F), jnp.float32),
    )(x, W)


def _tc_scale(xw, dp):
    def body(xw_ref, dp_ref, o_ref):
        o_ref[...] = _dinv(dp_ref[...]) * xw_ref[...]

    return pl.pallas_call(
        body, out_shape=jax.ShapeDtypeStruct((Z_ROWS, F), jnp.float32),
    )(xw, dp)


def _tc_mid(zp, y, b, W, dp):
    def body(zp_ref, y_ref, b_ref, w_ref, dp_ref, o_ref):
        dinv = _dinv(dp_ref[...])
        h = dinv * (zp_ref[0] + zp_ref[1] + y_ref[...]) + b_ref[...]
        h = jnp.where(h >= 0, h, h * SLOPE)
        o_ref[...] = dinv * _dot(h, w_ref[...])

    return pl.pallas_call(
        body, out_shape=jax.ShapeDtypeStruct((Z_ROWS, F), jnp.float32),
    )(zp, y, b, W, dp)


def _tc_final(zp, y, b, lw, lb, dp):
    def body(zp_ref, y_ref, b_ref, lw_ref, lb_ref, dp_ref, o_ref):
        dinv = _dinv(dp_ref[...])
        h = dinv * (zp_ref[0] + zp_ref[1] + y_ref[...]) + b_ref[...]
        h = jnp.where(h >= 0, h, h * SLOPE)
        o_ref[...] = jax.nn.sigmoid(_dot(h, lw_ref[...]) + lb_ref[...])

    return pl.pallas_call(
        body, out_shape=jax.ShapeDtypeStruct((Z_ROWS, OUT_F), jnp.float32),
    )(zp, y, b, lw, lb, dp)


def kernel(x, edge_index, W1, b1, W2, b2, lin_W, lin_b):
    src = edge_index[0].astype(jnp.int32)
    dst = edge_index[1].astype(jnp.int32)
    pad = N + jnp.arange(E_PAD - E, dtype=jnp.int32) % (Z_ROWS - N)
    src_p = jnp.concatenate([src, pad]).reshape(TOT_CH, CHUNK)
    dst_p = jnp.concatenate([dst, pad]).reshape(TOT_CH, CHUNK)
    idx_hbm = jnp.stack([src_p, dst_p], axis=1)

    x_p = jnp.pad(x, ((0, Z_ROWS - N), (0, 0)))
    zeros128 = jnp.zeros((ZB, F), jnp.float32)
    b1r = b1.reshape(1, F)
    b2r = b2.reshape(1, F)
    lbr = lin_b.reshape(1, OUT_F)

    dpc = _sc_deg(idx_hbm).reshape(NC, Z_ROWS, 1)
    xw1 = _tc_xw(x_p, W1)
    y1 = _tc_scale(xw1, dpc)
    zp1 = _sc_scatter(y1, idx_hbm, zeros128)
    y2 = _tc_mid(zp1, y1, b1r, W2, dpc)
    zp2 = _sc_scatter(y2, idx_hbm, zeros128)
    out = _tc_final(zp2, y2, b2r, lin_W, lbr, dpc)
    return out[:N]
